# Initial kernel scaffold; baseline (speedup 1.0000x reference)
import jax
import jax.numpy as jnp
from jax.experimental import pallas as pl


def kernel(input):
    raise NotImplementedError("write your pallas kernel here")


# speedup vs baseline: 1.8781x; 1.8781x over previous
import functools

import jax
import jax.numpy as jnp
from jax.experimental import pallas as pl
from jax.experimental.pallas import tpu as pltpu

K = 64


def _topk_body(x_ref, vals_ref, idx_ref, scratch, *, rows, n):
    scratch[...] = x_ref[...]

    def body(i, carry):
        vals, idxs = carry
        x = scratch[...]
        lane = jax.lax.broadcasted_iota(jnp.int32, (rows, n), 1)
        v = jnp.max(x, axis=1, keepdims=True)
        ind = jnp.min(jnp.where(x == v, lane, n), axis=1, keepdims=True)
        scratch[...] = jnp.where(lane == ind, -jnp.inf, x)
        kcol = jax.lax.broadcasted_iota(jnp.int32, (rows, K), 1)
        vals = jnp.where(kcol == i, v, vals)
        idxs = jnp.where(kcol == i, ind, idxs)
        return vals, idxs

    vals0 = jnp.full((rows, K), -jnp.inf, dtype=jnp.float32)
    idxs0 = jnp.zeros((rows, K), dtype=jnp.int32)
    vals, idxs = jax.lax.fori_loop(0, K, body, (vals0, idxs0))
    vals_ref[...] = vals
    idx_ref[...] = idxs


def kernel(input):
    rows, n = input.shape
    body = functools.partial(_topk_body, rows=rows, n=n)
    vals, idxs = pl.pallas_call(
        body,
        out_shape=(
            jax.ShapeDtypeStruct((rows, K), input.dtype),
            jax.ShapeDtypeStruct((rows, K), jnp.int32),
        ),
        scratch_shapes=[pltpu.VMEM((rows, n), input.dtype)],
    )(input)
    return (vals, idxs)


# speedup vs baseline: 2.9479x; 1.5696x over previous
import functools

import jax
import jax.numpy as jnp
from jax.experimental import pallas as pl
from jax.experimental.pallas import tpu as pltpu

K = 64
NEG = float("-inf")


def _topk_body(xt_ref, vals_ref, idx_ref, *, n, rows, q):
    p = n // q
    shiftq = q.bit_length() - 1
    pos_e = jax.lax.broadcasted_iota(jnp.int32, (q, rows), 0)

    def page_upd(pg, carry):
        s1, s2, s3, e1, e2, e3 = carry
        v = xt_ref[pl.ds(pg * q, q), :]
        ev = pg * q + pos_e
        b1 = v > s1
        b2 = v > s2
        b3 = v > s3
        s3n = jnp.where(b2, s2, jnp.where(b3, v, s3))
        e3n = jnp.where(b2, e2, jnp.where(b3, ev, e3))
        s2n = jnp.where(b1, s1, jnp.where(b2, v, s2))
        e2n = jnp.where(b1, e1, jnp.where(b2, ev, e2))
        s1n = jnp.where(b1, v, s1)
        e1n = jnp.where(b1, ev, e1)
        return s1n, s2n, s3n, e1n, e2n, e3n

    neg = jnp.full((q, rows), NEG, dtype=jnp.float32)
    zero = jnp.zeros((q, rows), dtype=jnp.int32)
    s1, s2, s3, e1, e2, e3 = jax.lax.fori_loop(
        0, p, page_upd, (neg, neg, neg, zero, zero, zero))

    krow = jax.lax.broadcasted_iota(jnp.int32, (K, rows), 0)
    big = jnp.int32(2 ** 30)

    def extract(k, carry):
        m, e, cnt, rbit, vals_t, idx_t = carry
        v = jnp.max(m, axis=0, keepdims=True)
        eq = m == v
        emin = jnp.min(jnp.where(eq, e, big), axis=0, keepdims=True)
        posmask = e == emin
        vals_t = jnp.where(krow == k, v, vals_t)
        idx_t = jnp.where(krow == k, emin, idx_t)
        bit = jnp.left_shift(jnp.int32(1), jnp.right_shift(emin, shiftq))
        rbit = jnp.where(posmask, rbit | bit, rbit)
        nv = jnp.where(cnt == 0, s2, jnp.where(cnt == 1, s3, NEG))
        ne = jnp.where(cnt == 0, e2, jnp.where(cnt == 1, e3, -1))
        m = jnp.where(posmask, nv, m)
        e = jnp.where(posmask, ne, e)
        cnt = jnp.where(posmask, cnt + 1, cnt)
        needs = posmask & (cnt >= 3)
        flag = jnp.max(needs.astype(jnp.int32)) > 0

        def recompute(args):
            m_, e_, rbit_, needs_ = args
            best = jnp.full((q, rows), NEG, dtype=jnp.float32)
            best_e = jnp.full((q, rows), -1, dtype=jnp.int32)
            for pg in range(p):
                val = xt_ref[pl.ds(pg * q, q), :]
                alive = (jnp.right_shift(rbit_, pg) & 1) == 0
                better = alive & (val > best)
                best = jnp.where(better, val, best)
                best_e = jnp.where(better, pg * q + pos_e, best_e)
            return (jnp.where(needs_, best, m_),
                    jnp.where(needs_, best_e, e_))

        def keep(args):
            m_, e_, _, _ = args
            return m_, e_

        m, e = jax.lax.cond(flag, recompute, keep, (m, e, rbit, needs))
        return m, e, cnt, rbit, vals_t, idx_t

    vals_t0 = jnp.full((K, rows), NEG, dtype=jnp.float32)
    idx_t0 = jnp.zeros((K, rows), dtype=jnp.int32)
    cnt0 = jnp.zeros((q, rows), dtype=jnp.int32)
    _, _, _, _, vals_t, idx_t = jax.lax.fori_loop(
        0, K, extract, (s1, e1, cnt0, zero, vals_t0, idx_t0))
    vals_ref[...] = vals_t
    idx_ref[...] = idx_t


def kernel(input):
    rows, n = input.shape
    q = n // 32
    xt = input.T
    body = functools.partial(_topk_body, n=n, rows=rows, q=q)
    vals_t, idx_t = pl.pallas_call(
        body,
        out_shape=(
            jax.ShapeDtypeStruct((K, rows), input.dtype),
            jax.ShapeDtypeStruct((K, rows), jnp.int32),
        ),
    )(xt)
    return (vals_t.T, idx_t.T)


# speedup vs baseline: 3.7871x; 1.2847x over previous
import functools

import jax
import jax.numpy as jnp
from jax.experimental import pallas as pl
from jax.experimental.pallas import tpu as pltpu

K = 64
NEG = float("-inf")


def _topk_body(xt_ref, vals_ref, idx_ref, *, n, rows, q):
    p = n // q
    pos_e = jax.lax.broadcasted_iota(jnp.int32, (q, rows), 0)

    def page_upd(pg, carry):
        s1, s2, s3, s4, e1, e2, e3, e4 = carry
        v = xt_ref[pl.ds(pg * q, q), :]
        ev = pg * q + pos_e
        b1 = v > s1
        b2 = v > s2
        b3 = v > s3
        b4 = v > s4
        s4n = jnp.where(b3, s3, jnp.where(b4, v, s4))
        e4n = jnp.where(b3, e3, jnp.where(b4, ev, e4))
        s3n = jnp.where(b2, s2, jnp.where(b3, v, s3))
        e3n = jnp.where(b2, e2, jnp.where(b3, ev, e3))
        s2n = jnp.where(b1, s1, jnp.where(b2, v, s2))
        e2n = jnp.where(b1, e1, jnp.where(b2, ev, e2))
        s1n = jnp.where(b1, v, s1)
        e1n = jnp.where(b1, ev, e1)
        return s1n, s2n, s3n, s4n, e1n, e2n, e3n, e4n

    neg = jnp.full((q, rows), NEG, dtype=jnp.float32)
    zero = jnp.zeros((q, rows), dtype=jnp.int32)
    s1, s2, s3, s4, e1, e2, e3, e4 = jax.lax.fori_loop(
        0, p, page_upd, (neg, neg, neg, neg, zero, zero, zero, zero))

    krow = jax.lax.broadcasted_iota(jnp.int32, (K, rows), 0)
    big = jnp.int32(2 ** 30)

    def extract(k, carry):
        m, e, cnt, last_v, last_e, vals_t, idx_t = carry
        v = jnp.max(m, axis=0, keepdims=True)
        eq = m == v
        emin = jnp.min(jnp.where(eq, e, big), axis=0, keepdims=True)
        posmask = e == emin
        vals_t = jnp.where(krow == k, v, vals_t)
        idx_t = jnp.where(krow == k, emin, idx_t)
        last_v = jnp.where(posmask, v, last_v)
        last_e = jnp.where(posmask, emin, last_e)
        nv = jnp.where(cnt == 0, s2,
                       jnp.where(cnt == 1, s3,
                                 jnp.where(cnt == 2, s4, NEG)))
        ne = jnp.where(cnt == 0, e2,
                       jnp.where(cnt == 1, e3,
                                 jnp.where(cnt == 2, e4, -1)))
        m = jnp.where(posmask, nv, m)
        e = jnp.where(posmask, ne, e)
        cnt = jnp.where(posmask, cnt + 1, cnt)
        needs = posmask & (cnt >= 4)
        flag = jnp.max(needs.astype(jnp.int32)) > 0

        def recompute(args):
            m_, e_, lv_, le_, needs_ = args
            best = jnp.full((q, rows), NEG, dtype=jnp.float32)
            best_e = jnp.full((q, rows), -1, dtype=jnp.int32)
            for pg in range(p):
                val = xt_ref[pl.ds(pg * q, q), :]
                ev = pg * q + pos_e
                alive = (val < lv_) | ((val == lv_) & (ev > le_))
                better = alive & (val > best)
                best = jnp.where(better, val, best)
                best_e = jnp.where(better, ev, best_e)
            return (jnp.where(needs_, best, m_),
                    jnp.where(needs_, best_e, e_))

        def keep(args):
            m_, e_, _, _, _ = args
            return m_, e_

        m, e = jax.lax.cond(flag, recompute, keep, (m, e, last_v, last_e, needs))
        return m, e, cnt, last_v, last_e, vals_t, idx_t

    vals_t0 = jnp.full((K, rows), NEG, dtype=jnp.float32)
    idx_t0 = jnp.zeros((K, rows), dtype=jnp.int32)
    cnt0 = jnp.zeros((q, rows), dtype=jnp.int32)
    lv0 = jnp.full((q, rows), float("inf"), dtype=jnp.float32)
    le0 = jnp.full((q, rows), -1, dtype=jnp.int32)
    _, _, _, _, _, vals_t, idx_t = jax.lax.fori_loop(
        0, K, extract, (s1, e1, cnt0, lv0, le0, vals_t0, idx_t0))
    vals_ref[...] = vals_t
    idx_ref[...] = idx_t


def kernel(input):
    rows, n = input.shape
    q = n // 128
    xt = input.T
    body = functools.partial(_topk_body, n=n, rows=rows, q=q)
    vals_t, idx_t = pl.pallas_call(
        body,
        out_shape=(
            jax.ShapeDtypeStruct((K, rows), input.dtype),
            jax.ShapeDtypeStruct((K, rows), jnp.int32),
        ),
    )(xt)
    return (vals_t.T, idx_t.T)


# speedup vs baseline: 6.7146x; 1.7730x over previous
import functools

import jax
import jax.numpy as jnp
from jax.experimental import pallas as pl
from jax.experimental.pallas import tpu as pltpu

K = 64
NEG = float("-inf")
DEPTH = 6


def _phase_a(x_ref, pos_e, n, rows, q):
    p = n // q

    def page_upd(pg, carry):
        s, e = carry
        v = x_ref[:, pl.ds(pg * q, q)]
        ev = pg * q + pos_e
        b = [v > s[i] for i in range(DEPTH)]
        sn = [None] * DEPTH
        en = [None] * DEPTH
        for i in range(DEPTH - 1, 0, -1):
            sn[i] = jnp.where(b[i - 1], s[i - 1], jnp.where(b[i], v, s[i]))
            en[i] = jnp.where(b[i - 1], e[i - 1], jnp.where(b[i], ev, e[i]))
        sn[0] = jnp.where(b[0], v, s[0])
        en[0] = jnp.where(b[0], ev, e[0])
        return tuple(sn), tuple(en)

    neg = jnp.full((rows, q), NEG, dtype=jnp.float32)
    zero = jnp.zeros((rows, q), dtype=jnp.int32)
    return jax.lax.fori_loop(
        0, p, page_upd, ((neg,) * DEPTH, (zero,) * DEPTH))


def _next_cached(s, e, cnt, rows, q):
    nv = jnp.full((rows, q), NEG, dtype=jnp.float32)
    ne = jnp.full((rows, q), -1, dtype=jnp.int32)
    for i in range(DEPTH - 1, 0, -1):
        hit = cnt == (i - 1)
        nv = jnp.where(hit, s[i], nv)
        ne = jnp.where(hit, e[i], ne)
    return nv, ne


def _fast_body(x_ref, vals_ref, idx_ref, flag_ref, *, n, rows, q):
    pos_e = jax.lax.broadcasted_iota(jnp.int32, (rows, q), 1)
    s, e = _phase_a(x_ref, pos_e, n, rows, q)
    kcol = jax.lax.broadcasted_iota(jnp.int32, (rows, K), 1)
    big = jnp.int32(2 ** 30)
    zero = jnp.zeros((rows, q), dtype=jnp.int32)

    def extract_fast(k, carry):
        m, em, cnt, ever, vals, idx = carry
        v = jnp.max(m, axis=1, keepdims=True)
        eq = m == v
        emin = jnp.min(jnp.where(eq, em, big), axis=1, keepdims=True)
        posmask = em == emin
        vals = jnp.where(kcol == k, v, vals)
        idx = jnp.where(kcol == k, emin, idx)
        nv, ne = _next_cached(s, e, cnt, rows, q)
        m = jnp.where(posmask, nv, m)
        em = jnp.where(posmask, ne, em)
        cnt = jnp.where(posmask, cnt + 1, cnt)
        ever = ever | jnp.where(posmask & (cnt >= DEPTH), 1, 0)
        return m, em, cnt, ever, vals, idx

    vals0 = jnp.full((rows, K), NEG, dtype=jnp.float32)
    idx0 = jnp.zeros((rows, K), dtype=jnp.int32)
    fzero = jnp.zeros((rows, q), dtype=jnp.int32)
    _, _, _, ever, vals_f, idx_f = jax.lax.fori_loop(
        0, K, extract_fast, (s[0], e[0], zero, fzero, vals0, idx0))
    vals_ref[...] = vals_f
    idx_ref[...] = idx_f
    f = jnp.max(ever, axis=1, keepdims=True)
    flag_ref[...] = jnp.max(f, axis=0, keepdims=True)


def _slow_body(x_ref, vals_ref, idx_ref, *, n, rows, q):
    p = n // q
    pos_e = jax.lax.broadcasted_iota(jnp.int32, (rows, q), 1)
    s, e = _phase_a(x_ref, pos_e, n, rows, q)
    kcol = jax.lax.broadcasted_iota(jnp.int32, (rows, K), 1)
    big = jnp.int32(2 ** 30)
    zero = jnp.zeros((rows, q), dtype=jnp.int32)

    def extract_slow(k, carry):
        m, em, cnt, last_v, last_e, vals, idx = carry
        v = jnp.max(m, axis=1, keepdims=True)
        eq = m == v
        emin = jnp.min(jnp.where(eq, em, big), axis=1, keepdims=True)
        posmask = em == emin
        vals = jnp.where(kcol == k, v, vals)
        idx = jnp.where(kcol == k, emin, idx)
        last_v = jnp.where(posmask, v, last_v)
        last_e = jnp.where(posmask, emin, last_e)
        nv, ne = _next_cached(s, e, cnt, rows, q)
        m = jnp.where(posmask, nv, m)
        em = jnp.where(posmask, ne, em)
        cnt = jnp.where(posmask, cnt + 1, cnt)
        needs = posmask & (cnt >= DEPTH)
        nflag = jnp.max(needs.astype(jnp.int32)) > 0

        def recompute(args):
            m2, em2, lv, le, need = args
            best = jnp.full((rows, q), NEG, dtype=jnp.float32)
            best_e = jnp.full((rows, q), -1, dtype=jnp.int32)
            for pg in range(p):
                val = x_ref[:, pl.ds(pg * q, q)]
                ev = pg * q + pos_e
                alive = (val < lv) | ((val == lv) & (ev > le))
                better = alive & (val > best)
                best = jnp.where(better, val, best)
                best_e = jnp.where(better, ev, best_e)
            return (jnp.where(need, best, m2),
                    jnp.where(need, best_e, em2))

        def keep(args):
            m2, em2, _, _, _ = args
            return m2, em2

        m, em = jax.lax.cond(
            nflag, recompute, keep, (m, em, last_v, last_e, needs))
        return m, em, cnt, last_v, last_e, vals, idx

    vals0 = jnp.full((rows, K), NEG, dtype=jnp.float32)
    idx0 = jnp.zeros((rows, K), dtype=jnp.int32)
    inf_ = jnp.full((rows, q), float("inf"), dtype=jnp.float32)
    negone = jnp.full((rows, q), -1, dtype=jnp.int32)
    _, _, _, _, _, vals_s, idx_s = jax.lax.fori_loop(
        0, K, extract_slow, (s[0], e[0], zero, inf_, negone, vals0, idx0))
    vals_ref[...] = vals_s
    idx_ref[...] = idx_s


def kernel(input):
    rows, n = input.shape
    q = n // 128
    fast = functools.partial(_fast_body, n=n, rows=rows, q=q)
    vals, idx, flag = pl.pallas_call(
        fast,
        out_shape=(
            jax.ShapeDtypeStruct((rows, K), input.dtype),
            jax.ShapeDtypeStruct((rows, K), jnp.int32),
            jax.ShapeDtypeStruct((1, 1), jnp.int32),
        ),
    )(input)

    def slow_path(_):
        slow = functools.partial(_slow_body, n=n, rows=rows, q=q)
        return pl.pallas_call(
            slow,
            out_shape=(
                jax.ShapeDtypeStruct((rows, K), input.dtype),
                jax.ShapeDtypeStruct((rows, K), jnp.int32),
            ),
        )(input)

    def fast_path(_):
        return vals, idx

    vals, idx = jax.lax.cond(flag[0, 0] > 0, slow_path, fast_path, None)
    return (vals, idx)


# speedup vs baseline: 6.7247x; 1.0015x over previous
import functools

import jax
import jax.numpy as jnp
from jax.experimental import pallas as pl
from jax.experimental.pallas import tpu as pltpu

K = 64
NEG = float("-inf")
DEPTH = 8


def _phase_a(x_ref, pos_e, n, rows, q):
    p = n // q

    def page_upd(pg, carry):
        s, e = carry
        v = x_ref[:, pl.ds(pg * q, q)]
        ev = pg * q + pos_e
        b = [v > s[i] for i in range(DEPTH)]
        sn = [None] * DEPTH
        en = [None] * DEPTH
        for i in range(DEPTH - 1, 0, -1):
            sn[i] = jnp.where(b[i - 1], s[i - 1], jnp.where(b[i], v, s[i]))
            en[i] = jnp.where(b[i - 1], e[i - 1], jnp.where(b[i], ev, e[i]))
        sn[0] = jnp.where(b[0], v, s[0])
        en[0] = jnp.where(b[0], ev, e[0])
        return tuple(sn), tuple(en)

    neg = jnp.full((rows, q), NEG, dtype=jnp.float32)
    zero = jnp.zeros((rows, q), dtype=jnp.int32)
    return jax.lax.fori_loop(
        0, p, page_upd, ((neg,) * DEPTH, (zero,) * DEPTH))


def _next_cached(s, e, cnt, rows, q):
    nv = jnp.full((rows, q), NEG, dtype=jnp.float32)
    ne = jnp.full((rows, q), -1, dtype=jnp.int32)
    for i in range(DEPTH - 1, 0, -1):
        hit = cnt == (i - 1)
        nv = jnp.where(hit, s[i], nv)
        ne = jnp.where(hit, e[i], ne)
    return nv, ne


def _fast_body(x_ref, vals_ref, idx_ref, flag_ref, *, n, rows, q):
    pos_e = jax.lax.broadcasted_iota(jnp.int32, (rows, q), 1)
    s, e = _phase_a(x_ref, pos_e, n, rows, q)
    kcol = jax.lax.broadcasted_iota(jnp.int32, (rows, K), 1)
    big = jnp.int32(2 ** 30)
    zero = jnp.zeros((rows, q), dtype=jnp.int32)

    def extract_fast(k, carry):
        m, em, cnt, ever, vals, idx = carry
        v = jnp.max(m, axis=1, keepdims=True)
        eq = m == v
        emin = jnp.min(jnp.where(eq, em, big), axis=1, keepdims=True)
        posmask = em == emin
        vals = jnp.where(kcol == k, v, vals)
        idx = jnp.where(kcol == k, emin, idx)
        nv, ne = _next_cached(s, e, cnt, rows, q)
        m = jnp.where(posmask, nv, m)
        em = jnp.where(posmask, ne, em)
        cnt = jnp.where(posmask, cnt + 1, cnt)
        ever = ever | jnp.where(posmask & (cnt >= DEPTH), 1, 0)
        return m, em, cnt, ever, vals, idx

    vals0 = jnp.full((rows, K), NEG, dtype=jnp.float32)
    idx0 = jnp.zeros((rows, K), dtype=jnp.int32)
    fzero = jnp.zeros((rows, q), dtype=jnp.int32)
    _, _, _, ever, vals_f, idx_f = jax.lax.fori_loop(
        0, K, extract_fast, (s[0], e[0], zero, fzero, vals0, idx0))
    vals_ref[...] = vals_f
    idx_ref[...] = idx_f
    f = jnp.max(ever, axis=1, keepdims=True)
    flag_ref[...] = jnp.max(f, axis=0, keepdims=True)


def _slow_body(x_ref, vals_ref, idx_ref, *, n, rows, q):
    p = n // q
    pos_e = jax.lax.broadcasted_iota(jnp.int32, (rows, q), 1)
    s, e = _phase_a(x_ref, pos_e, n, rows, q)
    kcol = jax.lax.broadcasted_iota(jnp.int32, (rows, K), 1)
    big = jnp.int32(2 ** 30)
    zero = jnp.zeros((rows, q), dtype=jnp.int32)

    def extract_slow(k, carry):
        m, em, cnt, last_v, last_e, vals, idx = carry
        v = jnp.max(m, axis=1, keepdims=True)
        eq = m == v
        emin = jnp.min(jnp.where(eq, em, big), axis=1, keepdims=True)
        posmask = em == emin
        vals = jnp.where(kcol == k, v, vals)
        idx = jnp.where(kcol == k, emin, idx)
        last_v = jnp.where(posmask, v, last_v)
        last_e = jnp.where(posmask, emin, last_e)
        nv, ne = _next_cached(s, e, cnt, rows, q)
        m = jnp.where(posmask, nv, m)
        em = jnp.where(posmask, ne, em)
        cnt = jnp.where(posmask, cnt + 1, cnt)
        needs = posmask & (cnt >= DEPTH)
        nflag = jnp.max(needs.astype(jnp.int32)) > 0

        def recompute(args):
            m2, em2, lv, le, need = args
            best = jnp.full((rows, q), NEG, dtype=jnp.float32)
            best_e = jnp.full((rows, q), -1, dtype=jnp.int32)
            for pg in range(p):
                val = x_ref[:, pl.ds(pg * q, q)]
                ev = pg * q + pos_e
                alive = (val < lv) | ((val == lv) & (ev > le))
                better = alive & (val > best)
                best = jnp.where(better, val, best)
                best_e = jnp.where(better, ev, best_e)
            return (jnp.where(need, best, m2),
                    jnp.where(need, best_e, em2))

        def keep(args):
            m2, em2, _, _, _ = args
            return m2, em2

        m, em = jax.lax.cond(
            nflag, recompute, keep, (m, em, last_v, last_e, needs))
        return m, em, cnt, last_v, last_e, vals, idx

    vals0 = jnp.full((rows, K), NEG, dtype=jnp.float32)
    idx0 = jnp.zeros((rows, K), dtype=jnp.int32)
    inf_ = jnp.full((rows, q), float("inf"), dtype=jnp.float32)
    negone = jnp.full((rows, q), -1, dtype=jnp.int32)
    _, _, _, _, _, vals_s, idx_s = jax.lax.fori_loop(
        0, K, extract_slow, (s[0], e[0], zero, inf_, negone, vals0, idx0))
    vals_ref[...] = vals_s
    idx_ref[...] = idx_s


def kernel(input):
    rows, n = input.shape
    q = n // 256
    fast = functools.partial(_fast_body, n=n, rows=rows, q=q)
    vals, idx, flag = pl.pallas_call(
        fast,
        out_shape=(
            jax.ShapeDtypeStruct((rows, K), input.dtype),
            jax.ShapeDtypeStruct((rows, K), jnp.int32),
            jax.ShapeDtypeStruct((1, 1), jnp.int32),
        ),
    )(input)

    def slow_path(_):
        slow = functools.partial(_slow_body, n=n, rows=rows, q=q)
        return pl.pallas_call(
            slow,
            out_shape=(
                jax.ShapeDtypeStruct((rows, K), input.dtype),
                jax.ShapeDtypeStruct((rows, K), jnp.int32),
            ),
        )(input)

    def fast_path(_):
        return vals, idx

    vals, idx = jax.lax.cond(flag[0, 0] > 0, slow_path, fast_path, None)
    return (vals, idx)
